# baseline (speedup 1.0000x reference)
import functools

import jax
import jax.numpy as jnp
from jax.experimental import pallas as pl

N = 10000
D = 256
H = 128
NP = 10240


def _make_tc_layer(nrows, block, relu, split_out):
  grid = (nrows // block,)

  def body(h0, h1, s0, s1, dg, wst, wat, b, *outs):
    acc = jnp.dot(h0[...], wst[0], preferred_element_type=jnp.float32)
    acc += jnp.dot(h1[...], wst[1], preferred_element_type=jnp.float32)
    sacc = jnp.dot(s0[...], wat[0], preferred_element_type=jnp.float32)
    sacc += jnp.dot(s1[...], wat[1], preferred_element_type=jnp.float32)
    d = jnp.maximum(dg[...], 1.0)
    r = acc + sacc / d + b[...]
    if relu:
      r = jnp.maximum(r, 0.0)
    if split_out:
      i = pl.program_id(0)
      gr = i * block + jax.lax.broadcasted_iota(jnp.int32, r.shape, 0)
      r = jnp.where(gr < N, r, 0.0)
      outs[0][0] = r[:, :H]
      outs[0][1] = r[:, H:]
    else:
      outs[0][...] = r

  row_spec_h = pl.BlockSpec((block, H), lambda i: (i, 0))
  in_specs = [
      row_spec_h, row_spec_h, row_spec_h, row_spec_h,
      pl.BlockSpec((block, 1), lambda i: (i, 0)),
      pl.BlockSpec((2, H, D), lambda i: (0, 0, 0)),
      pl.BlockSpec((2, H, D), lambda i: (0, 0, 0)),
      pl.BlockSpec((1, D), lambda i: (0, 0)),
  ]
  if split_out:
    out_shape = [jax.ShapeDtypeStruct((2, nrows, H), jnp.float32)]
    out_specs = [pl.BlockSpec((2, block, H), lambda i: (0, i, 0))]
  else:
    out_shape = [jax.ShapeDtypeStruct((nrows, D), jnp.float32)]
    out_specs = [pl.BlockSpec((block, D), lambda i: (i, 0))]

  return pl.pallas_call(
      body, grid=grid, in_specs=in_specs, out_specs=out_specs,
      out_shape=out_shape)


_make_tc_layer = functools.lru_cache(maxsize=None)(_make_tc_layer)


def _segsum_halves(h, src, dst):
  msg = jnp.take(h, src, axis=0)
  s = jax.ops.segment_sum(msg, dst, num_segments=N)
  zp = jnp.zeros((NP - N, H), jnp.float32)
  return (jnp.concatenate([s[:, :H], zp], axis=0),
          jnp.concatenate([s[:, H:], zp], axis=0))


def kernel(x, edge_index, W0, b0, W1, b1):
  ei = edge_index.astype(jnp.int32)
  src, dst = ei[0], ei[1]

  zp = jnp.zeros((NP - N, H), jnp.float32)
  xp0 = jnp.concatenate([x[:, :H], zp], axis=0)
  xp1 = jnp.concatenate([x[:, H:], zp], axis=0)

  ones = jnp.ones((src.shape[0],), jnp.float32)
  deg = jax.ops.segment_sum(ones, dst, num_segments=N)
  deg = jnp.concatenate([deg, jnp.zeros((NP - N,), jnp.float32)])[:, None]

  wst0 = W0[:, :D].T.reshape(2, H, D)
  wat0 = W0[:, D:].T.reshape(2, H, D)
  wst1 = W1[:, :D].T.reshape(2, H, D)
  wat1 = W1[:, D:].T.reshape(2, H, D)

  tc_layer0 = _make_tc_layer(NP, 1024, relu=True, split_out=True)
  tc_layer1 = _make_tc_layer(NP, 1024, relu=False, split_out=False)

  s0, s1 = _segsum_halves(x, src, dst)
  h2 = tc_layer0(xp0, xp1, s0, s1, deg, wst0, wat0, b0.reshape(1, D))[0]
  h_full = jnp.concatenate([h2[0][:N], h2[1][:N]], axis=1)
  t0, t1 = _segsum_halves(h_full, src, dst)
  out = tc_layer1(h2[0], h2[1], t0, t1, deg, wst1, wat1,
                  b1.reshape(1, D))[0]
  return out[:N]
